# baseline (speedup 1.0000x reference)
import functools

import jax
import jax.numpy as jnp
from jax import lax
from jax.experimental import pallas as pl
from jax.experimental.pallas import tpu as pltpu
from jax.experimental.pallas import tpu_sc as plsc

B = 16384
HID = 128
NC = 2
NS = 16
NW = NC * NS
BPW = B // NW
CH = 128
NCH = BPW // CH

_sc_mesh = plsc.VectorSubcoreMesh(core_axis_name="c", subcore_axis_name="s")


NBUF = 4
LOOKAHEAD = 2


def _sc_gather_body(mid_h, sing_h, gen_h, emus_h, esing_h, egen_h,
                    out_m, out_s, out_g, idx_v, bufs, sems):
    wid = lax.axis_index("s") * NC + lax.axis_index("c")
    for t, idx_h in enumerate((mid_h, sing_h, gen_h)):
        pltpu.sync_copy(idx_h.at[pl.ds(wid * NCH, NCH)],
                        idx_v.at[pl.ds(t * NCH, NCH)])
    chunks = [(tab_h, out_h, t * NCH + j, j)
              for t, (tab_h, out_h) in enumerate(((emus_h, out_m),
                                                  (esing_h, out_s),
                                                  (egen_h, out_g)))
              for j in range(NCH)]
    n = len(chunks)
    gh = {}
    wh = {}
    for k in range(n + LOOKAHEAD):
        if k < n:
            b = k % NBUF
            if k >= NBUF:
                wh[k - NBUF].wait()
            tab_h, _, ir, _ = chunks[k]
            gh[k] = pltpu.async_copy(tab_h.at[idx_v.at[ir]],
                                     bufs.at[b], sems.at[b])
        kp = k - LOOKAHEAD
        if kp >= 0:
            b2 = kp % NBUF
            gh[kp].wait()
            _, out_h, _, j = chunks[kp]
            wh[kp] = pltpu.async_copy(
                bufs.at[b2], out_h.at[pl.ds(wid * BPW + j * CH, CH)],
                sems.at[b2])
    for k in range(n - NBUF, n):
        wh[k].wait()


@functools.partial(
    pl.kernel,
    out_type=[jax.ShapeDtypeStruct((B, HID), jnp.float32)] * 3,
    mesh=_sc_mesh,
    scratch_types=[
        pltpu.VMEM((3 * NCH, CH), jnp.int32),
        pltpu.VMEM((NBUF, CH, HID), jnp.float32),
        pltpu.SemaphoreType.DMA((NBUF,)),
    ],
)
def _sc_gather(*args):
    _sc_gather_body(*args)


def _tc_body(feat_ref, memb_ref, sing_ref, gen_ref,
             wf_ref, bf_ref, wout_ref, bo_ref, out_ref):
    f = jnp.dot(feat_ref[:], wf_ref[:], preferred_element_type=jnp.float32)
    f = f + bf_ref[:]
    acc = jnp.dot(memb_ref[:], wout_ref[0:HID, :],
                  preferred_element_type=jnp.float32)
    acc = acc + jnp.dot(f, wout_ref[HID:2 * HID, :],
                        preferred_element_type=jnp.float32)
    acc = acc + jnp.dot(sing_ref[:], wout_ref[2 * HID:3 * HID, :],
                        preferred_element_type=jnp.float32)
    acc = acc + jnp.dot(gen_ref[:], wout_ref[3 * HID:4 * HID, :],
                        preferred_element_type=jnp.float32)
    out_ref[:] = acc + bo_ref[:]


def kernel(features, lyric, singer, genre, mid,
           W_feat, b_feat, E_sing, E_gen, E_mus, W_out, b_out):
    del lyric
    mid_i = mid.astype(jnp.int32).reshape(B // CH, CH)
    sing_i = singer.astype(jnp.int32).reshape(B // CH, CH)
    gen_i = genre.astype(jnp.int32).reshape(B // CH, CH)

    memb, sing, gen = _sc_gather(mid_i, sing_i, gen_i, E_mus, E_sing, E_gen)

    BLK = 1024
    grid = (B // BLK,)
    row_spec = pl.BlockSpec((BLK, HID), lambda i: (i, 0))
    out = pl.pallas_call(
        _tc_body,
        grid=grid,
        in_specs=[
            row_spec,
            row_spec,
            row_spec,
            row_spec,
            pl.BlockSpec((HID, HID), lambda i: (0, 0)),
            pl.BlockSpec((1, HID), lambda i: (0, 0)),
            pl.BlockSpec((4 * HID, 2 * HID), lambda i: (0, 0)),
            pl.BlockSpec((1, 2 * HID), lambda i: (0, 0)),
        ],
        out_specs=pl.BlockSpec((BLK, 2 * HID), lambda i: (i, 0)),
        out_shape=jax.ShapeDtypeStruct((B, 2 * HID), jnp.float32),
    )(features, memb, sing, gen,
      W_feat, b_feat.reshape(1, HID), W_out, b_out.reshape(1, 2 * HID))
    return out


# speedup vs baseline: 1.9576x; 1.9576x over previous
import functools

import jax
import jax.numpy as jnp
from jax import lax
from jax.experimental import pallas as pl
from jax.experimental.pallas import tpu as pltpu
from jax.experimental.pallas import tpu_sc as plsc

B = 16384
HID = 128
N_SING = 417
N_GEN = 18
SING_PAD = 512
GEN_PAD = 128

NC = 2
NS = 16
NW = NC * NS
BPW = B // NW
CH = 128
NCH = BPW // CH
LOOKAHEAD = 2

_sc_mesh = plsc.VectorSubcoreMesh(core_axis_name="c", subcore_axis_name="s")


def _sc_gather_body(mid_h, emus_h, out_m, idx_v, bufs, sems):
    wid = lax.axis_index("s") * NC + lax.axis_index("c")
    pltpu.sync_copy(mid_h.at[pl.ds(wid * NCH, NCH)], idx_v)
    gh = {}
    wh = {}
    for k in range(NCH + LOOKAHEAD):
        if k < NCH:
            gh[k] = pltpu.async_copy(emus_h.at[idx_v.at[k]],
                                     bufs.at[k], sems.at[k])
        kp = k - LOOKAHEAD
        if kp >= 0:
            gh[kp].wait()
            wh[kp] = pltpu.async_copy(
                bufs.at[kp], out_m.at[pl.ds(wid * BPW + kp * CH, CH)],
                sems.at[kp])
    for k in range(NCH):
        wh[k].wait()


@functools.partial(
    pl.kernel,
    out_type=jax.ShapeDtypeStruct((B, HID), jnp.float32),
    mesh=_sc_mesh,
    scratch_types=[
        pltpu.VMEM((NCH, CH), jnp.int32),
        pltpu.VMEM((NCH, CH, HID), jnp.float32),
        pltpu.SemaphoreType.DMA((NCH,)),
    ],
)
def _sc_gather(*args):
    _sc_gather_body(*args)


def _tc_body(feat_ref, memb_ref, sing_idx_ref, gen_idx_ref,
             wf_ref, bf_ref, es_ref, eg_ref, wout_ref, bo_ref, out_ref):
    f = jnp.dot(feat_ref[:], wf_ref[:], preferred_element_type=jnp.float32)
    f = f + bf_ref[:]
    blk = feat_ref.shape[0]
    oh_s = (sing_idx_ref[:] ==
            lax.broadcasted_iota(jnp.int32, (blk, SING_PAD), 1)
            ).astype(jnp.float32)
    oh_g = (gen_idx_ref[:] ==
            lax.broadcasted_iota(jnp.int32, (blk, GEN_PAD), 1)
            ).astype(jnp.float32)
    sing = jnp.dot(oh_s, es_ref[:], preferred_element_type=jnp.float32)
    gen = jnp.dot(oh_g, eg_ref[:], preferred_element_type=jnp.float32)
    acc = jnp.dot(memb_ref[:], wout_ref[0:HID, :],
                  preferred_element_type=jnp.float32)
    acc = acc + jnp.dot(f, wout_ref[HID:2 * HID, :],
                        preferred_element_type=jnp.float32)
    acc = acc + jnp.dot(sing, wout_ref[2 * HID:3 * HID, :],
                        preferred_element_type=jnp.float32)
    acc = acc + jnp.dot(gen, wout_ref[3 * HID:4 * HID, :],
                        preferred_element_type=jnp.float32)
    out_ref[:] = acc + bo_ref[:]


def kernel(features, lyric, singer, genre, mid,
           W_feat, b_feat, E_sing, E_gen, E_mus, W_out, b_out):
    del lyric
    mid_i = mid.astype(jnp.int32).reshape(B // CH, CH)

    memb = _sc_gather(mid_i, E_mus)

    es_pad = jnp.zeros((SING_PAD, HID), jnp.float32).at[:N_SING].set(E_sing)
    eg_pad = jnp.zeros((GEN_PAD, HID), jnp.float32).at[:N_GEN].set(E_gen)

    BLK = 1024
    grid = (B // BLK,)
    row_spec = pl.BlockSpec((BLK, HID), lambda i: (i, 0))
    idx_spec = pl.BlockSpec((BLK, 1), lambda i: (i, 0))
    out = pl.pallas_call(
        _tc_body,
        grid=grid,
        in_specs=[
            row_spec,
            row_spec,
            idx_spec,
            idx_spec,
            pl.BlockSpec((HID, HID), lambda i: (0, 0)),
            pl.BlockSpec((1, HID), lambda i: (0, 0)),
            pl.BlockSpec((SING_PAD, HID), lambda i: (0, 0)),
            pl.BlockSpec((GEN_PAD, HID), lambda i: (0, 0)),
            pl.BlockSpec((4 * HID, 2 * HID), lambda i: (0, 0)),
            pl.BlockSpec((1, 2 * HID), lambda i: (0, 0)),
        ],
        out_specs=pl.BlockSpec((BLK, 2 * HID), lambda i: (i, 0)),
        out_shape=jax.ShapeDtypeStruct((B, 2 * HID), jnp.float32),
    )(features, memb,
      singer.astype(jnp.int32).reshape(B, 1),
      genre.astype(jnp.int32).reshape(B, 1),
      W_feat, b_feat.reshape(1, HID), es_pad, eg_pad,
      W_out, b_out.reshape(1, 2 * HID))
    return out
